# baseline (speedup 1.0000x reference)
import functools

import jax
import jax.numpy as jnp
from jax import lax
from jax.experimental import pallas as pl
from jax.experimental.pallas import tpu as pltpu
from jax.experimental.pallas import tpu_sc as plsc

B = 4
Z = 96
YX = 96
NW = 32
WPB = NW // B
ZPW = Z // WPB
LANES = 16
VPR = YX // LANES
NLAB = 64
BINS = NLAB * LANES
LPB = 2
TEMP = 0.2


def _sc_loss_body(pred_hbm, lab_hbm, out_hbm,
                  lab0, lab1, a0, a1, b0, b1, accs, accc,
                  wbs, wbc, s64, c64, obuf, sh_s, sh_c, sem0, sem1):
    cid = lax.axis_index("c")
    sid = lax.axis_index("s")
    wid = cid * 16 + sid
    b = wid // WPB
    lb = sid // WPB
    z0 = (wid % WPB) * ZPW

    iota = lax.iota(jnp.int32, LANES)
    zeros = jnp.zeros((LANES,), jnp.float32)

    @plsc.parallel_loop(0, NLAB, unroll=8)
    def _(j):
        accs[pl.ds(j * LANES, LANES)] = zeros
        accc[pl.ds(j * LANES, LANES)] = zeros

    labs = [lab0, lab1]
    avs = [a0, a1]
    bvs = [b0, b1]
    sems = [sem0, sem1]

    def start(k, bank):
        z = z0 + k
        return (
            pltpu.async_copy(lab_hbm.at[b, 0, z], labs[bank], sems[bank]),
            pltpu.async_copy(pred_hbm.at[b, 0, z], avs[bank], sems[bank]),
            pltpu.async_copy(pred_hbm.at[b, 1, z], bvs[bank], sems[bank]),
        )

    ones = jnp.ones((LANES,), jnp.float32)

    def compute(bank):
        labr, ar, br = labs[bank], avs[bank], bvs[bank]

        @plsc.parallel_loop(0, YX, unroll=12)
        def _(r):
            idxs, es = [], []
            for c in range(VPR):
                sl = pl.ds(c * LANES, LANES)
                li = labr[r, sl].astype(jnp.int32)
                idxs.append(li * LANES + iota)
            for c in range(VPR):
                sl = pl.ds(c * LANES, LANES)
                d = ar[r, sl] - br[r, sl]
                es.append(jnp.exp(d))
            fgs = [1.0 / (1.0 + e) for e in es]
            for c in range(VPR):
                plsc.addupdate_scatter(accs, [idxs[c]], fgs[c])
                plsc.addupdate_scatter(accc, [idxs[c]], ones)

    handles = [None, None]
    handles[0] = start(0, 0)
    for k in range(ZPW):
        bank = k % 2
        if k + 1 < ZPW:
            handles[1 - bank] = start(k + 1, 1 - bank)
        for h in handles[bank]:
            h.wait()
        compute(bank)

    for g in range(NLAB // LANES):
        base = iota * LANES + g * LANES * LANES
        sacc = jnp.zeros((LANES,), jnp.float32)
        cacc = jnp.zeros((LANES,), jnp.float32)
        for j in range(LANES):
            sacc = sacc + plsc.load_gather(accs, [base + j])
            cacc = cacc + plsc.load_gather(accc, [base + j])
        s64[pl.ds(g * LANES, LANES)] = sacc
        c64[pl.ds(g * LANES, LANES)] = cacc

    pltpu.sync_copy(s64, sh_s.at[sid])
    pltpu.sync_copy(c64, sh_c.at[sid])
    plsc.subcore_barrier()

    @pl.when(sid == 0)
    def _():
        pltpu.sync_copy(sh_s, wbs)
        pltpu.sync_copy(sh_c, wbc)
        total_c = jnp.float32(0.0)
        n_c = jnp.float32(0.0)
        for lbi in range(LPB):
            for g in range(NLAB // LANES):
                sl = pl.ds(g * LANES, LANES)
                ssum = jnp.zeros((LANES,), jnp.float32)
                csum = jnp.zeros((LANES,), jnp.float32)
                for w in range(WPB):
                    ssum = ssum + wbs[lbi * WPB + w, sl]
                    csum = csum + wbc[lbi * WPB + w, sl]
                s64[sl] = ssum
                c64[sl] = csum
            neg = jnp.float32(-1e30)
            ps, pres = [], []
            for st in (1, 2, 3):
                gidx = iota * 4 + st
                s_s = plsc.load_gather(s64, [gidx])
                c_s = plsc.load_gather(c64, [gidx])
                ps.append(s_s / jnp.maximum(c_s, 1.0))
                pres.append(jnp.logical_and(c_s >= 1.0, iota >= 1))
            zz = [jnp.where(pr, -pv / TEMP, neg) for pv, pr in zip(ps, pres)]
            m = jnp.maximum(zz[0], jnp.maximum(zz[1], zz[2]))
            es = [jnp.where(pr, jnp.exp(z - m), 0.0) for z, pr in zip(zz, pres)]
            den = es[0] + es[1] + es[2]
            num = ps[0] * es[0] + ps[1] * es[1] + ps[2] * es[2]
            score = num / jnp.maximum(den, jnp.float32(1e-30))
            nv = (pres[0].astype(jnp.float32) + pres[1].astype(jnp.float32)
                  + pres[2].astype(jnp.float32))
            valid = nv >= 2.0
            contrib = jnp.where(valid, 1.0 - score, 0.0)
            total_c = total_c + jnp.sum(contrib)
            n_c = n_c + jnp.sum(valid.astype(jnp.float32))
        vout = jnp.where(iota == 0, total_c,
                         jnp.where(iota == 1, n_c, 0.0))
        obuf[...] = vout
        pltpu.sync_copy(obuf, out_hbm.at[cid])


@jax.jit
def _sc_loss(pred, lab):
    mesh = plsc.VectorSubcoreMesh(core_axis_name="c", subcore_axis_name="s")
    f = functools.partial(
        pl.kernel,
        out_type=jax.ShapeDtypeStruct((2, LANES), jnp.float32),
        mesh=mesh,
        scratch_types=[
            pltpu.VMEM((YX, YX), jnp.float32),
            pltpu.VMEM((YX, YX), jnp.float32),
            pltpu.VMEM((YX, YX), jnp.float32),
            pltpu.VMEM((YX, YX), jnp.float32),
            pltpu.VMEM((YX, YX), jnp.float32),
            pltpu.VMEM((YX, YX), jnp.float32),
            pltpu.VMEM((BINS,), jnp.float32),
            pltpu.VMEM((BINS,), jnp.float32),
            pltpu.VMEM((16, 128), jnp.float32),
            pltpu.VMEM((16, 128), jnp.float32),
            pltpu.VMEM((128,), jnp.float32),
            pltpu.VMEM((128,), jnp.float32),
            pltpu.VMEM((LANES,), jnp.float32),
            pltpu.VMEM_SHARED((16, 128), jnp.float32),
            pltpu.VMEM_SHARED((16, 128), jnp.float32),
            pltpu.SemaphoreType.DMA,
            pltpu.SemaphoreType.DMA,
        ],
        compiler_params=pltpu.CompilerParams(
            needs_layout_passes=False, use_tc_tiling_on_sc=True
        ),
    )(_sc_loss_body)
    return f(pred, lab)


def kernel(pred, stub_label_map):
    parts = _sc_loss(pred, stub_label_map)
    total = parts[0, 0] + parts[1, 0]
    n = parts[0, 1] + parts[1, 1]
    return jnp.where(n > 0, total / jnp.maximum(n, 1.0), 0.0).reshape(())


# speedup vs baseline: 1.0719x; 1.0719x over previous
import functools

import jax
import jax.numpy as jnp
from jax import lax
from jax.experimental import pallas as pl
from jax.experimental.pallas import tpu as pltpu
from jax.experimental.pallas import tpu_sc as plsc

B = 4
Z = 96
YX = 96
NW = 32
WPB = NW // B
ZPW = Z // WPB
LANES = 16
VPR = YX // LANES
NLAB = 64
BINS = NLAB * LANES
LPB = 2
TEMP = 0.2


def _sc_loss_body(pred_hbm, lab_hbm, out_hbm,
                  lab0, lab1, a0, a1, b0, b1, accs, accc,
                  wbs, wbc, s64, c64, obuf, sh_s, sh_c, sem0, sem1):
    cid = lax.axis_index("c")
    sid = lax.axis_index("s")
    wid = cid * 16 + sid
    b = wid // WPB
    lb = sid // WPB
    z0 = (wid % WPB) * ZPW

    iota = lax.iota(jnp.int32, LANES)
    zeros = jnp.zeros((LANES,), jnp.float32)

    labs = [lab0, lab1]
    avs = [a0, a1]
    bvs = [b0, b1]
    sems = [sem0, sem1]

    def start(k, bank):
        z = z0 + k
        return (
            pltpu.async_copy(lab_hbm.at[b, 0, z], labs[bank], sems[bank]),
            pltpu.async_copy(pred_hbm.at[b, 0, z], avs[bank], sems[bank]),
            pltpu.async_copy(pred_hbm.at[b, 1, z], bvs[bank], sems[bank]),
        )

    ones = jnp.ones((LANES,), jnp.float32)

    def compute(bank):
        labr, ar, br = labs[bank], avs[bank], bvs[bank]

        @plsc.parallel_loop(0, YX, unroll=6)
        def _(r):
            idxs, es = [], []
            for c in range(VPR):
                sl = pl.ds(c * LANES, LANES)
                li = labr[r, sl].astype(jnp.int32)
                idxs.append(li * LANES + iota)
            for c in range(VPR):
                sl = pl.ds(c * LANES, LANES)
                d = ar[r, sl] - br[r, sl]
                es.append(jnp.exp(d))
            fgs = [1.0 / (1.0 + e) for e in es]
            for c in range(VPR):
                plsc.addupdate_scatter(accs, [idxs[c]], fgs[c])
                plsc.addupdate_scatter(accc, [idxs[c]], ones)

    handles = [None, None]
    handles[0] = start(0, 0)

    @plsc.parallel_loop(0, NLAB, unroll=8)
    def _(j):
        accs[pl.ds(j * LANES, LANES)] = zeros
        accc[pl.ds(j * LANES, LANES)] = zeros

    for k in range(ZPW):
        bank = k % 2
        if k + 1 < ZPW:
            handles[1 - bank] = start(k + 1, 1 - bank)
        for h in handles[bank]:
            h.wait()
        compute(bank)

    for g in range(NLAB // LANES):
        base = iota * LANES + g * LANES * LANES
        sacc = jnp.zeros((LANES,), jnp.float32)
        cacc = jnp.zeros((LANES,), jnp.float32)
        for j in range(LANES):
            sacc = sacc + plsc.load_gather(accs, [base + j])
            cacc = cacc + plsc.load_gather(accc, [base + j])
        s64[pl.ds(g * LANES, LANES)] = sacc
        c64[pl.ds(g * LANES, LANES)] = cacc

    pltpu.sync_copy(s64, sh_s.at[sid])
    pltpu.sync_copy(c64, sh_c.at[sid])
    plsc.subcore_barrier()

    @pl.when(sid == 0)
    def _():
        pltpu.sync_copy(sh_s, wbs)
        pltpu.sync_copy(sh_c, wbc)
        total_c = jnp.float32(0.0)
        n_c = jnp.float32(0.0)
        for lbi in range(LPB):
            for g in range(NLAB // LANES):
                sl = pl.ds(g * LANES, LANES)
                ssum = jnp.zeros((LANES,), jnp.float32)
                csum = jnp.zeros((LANES,), jnp.float32)
                for w in range(WPB):
                    ssum = ssum + wbs[lbi * WPB + w, sl]
                    csum = csum + wbc[lbi * WPB + w, sl]
                s64[sl] = ssum
                c64[sl] = csum
            neg = jnp.float32(-1e30)
            ps, pres = [], []
            for st in (1, 2, 3):
                gidx = iota * 4 + st
                s_s = plsc.load_gather(s64, [gidx])
                c_s = plsc.load_gather(c64, [gidx])
                ps.append(s_s / jnp.maximum(c_s, 1.0))
                pres.append(jnp.logical_and(c_s >= 1.0, iota >= 1))
            zz = [jnp.where(pr, -pv / TEMP, neg) for pv, pr in zip(ps, pres)]
            m = jnp.maximum(zz[0], jnp.maximum(zz[1], zz[2]))
            es = [jnp.where(pr, jnp.exp(z - m), 0.0) for z, pr in zip(zz, pres)]
            den = es[0] + es[1] + es[2]
            num = ps[0] * es[0] + ps[1] * es[1] + ps[2] * es[2]
            score = num / jnp.maximum(den, jnp.float32(1e-30))
            nv = (pres[0].astype(jnp.float32) + pres[1].astype(jnp.float32)
                  + pres[2].astype(jnp.float32))
            valid = nv >= 2.0
            contrib = jnp.where(valid, 1.0 - score, 0.0)
            total_c = total_c + jnp.sum(contrib)
            n_c = n_c + jnp.sum(valid.astype(jnp.float32))
        vout = jnp.where(iota == 0, total_c,
                         jnp.where(iota == 1, n_c, 0.0))
        obuf[...] = vout
        pltpu.sync_copy(obuf, out_hbm.at[cid])


@jax.jit
def _sc_loss(pred, lab):
    mesh = plsc.VectorSubcoreMesh(core_axis_name="c", subcore_axis_name="s")
    f = functools.partial(
        pl.kernel,
        out_type=jax.ShapeDtypeStruct((2, LANES), jnp.float32),
        mesh=mesh,
        scratch_types=[
            pltpu.VMEM((YX, YX), jnp.float32),
            pltpu.VMEM((YX, YX), jnp.float32),
            pltpu.VMEM((YX, YX), jnp.float32),
            pltpu.VMEM((YX, YX), jnp.float32),
            pltpu.VMEM((YX, YX), jnp.float32),
            pltpu.VMEM((YX, YX), jnp.float32),
            pltpu.VMEM((BINS,), jnp.float32),
            pltpu.VMEM((BINS,), jnp.float32),
            pltpu.VMEM((16, 128), jnp.float32),
            pltpu.VMEM((16, 128), jnp.float32),
            pltpu.VMEM((128,), jnp.float32),
            pltpu.VMEM((128,), jnp.float32),
            pltpu.VMEM((LANES,), jnp.float32),
            pltpu.VMEM_SHARED((16, 128), jnp.float32),
            pltpu.VMEM_SHARED((16, 128), jnp.float32),
            pltpu.SemaphoreType.DMA,
            pltpu.SemaphoreType.DMA,
        ],
        compiler_params=pltpu.CompilerParams(
            needs_layout_passes=False, use_tc_tiling_on_sc=True
        ),
    )(_sc_loss_body)
    return f(pred, lab)


def kernel(pred, stub_label_map):
    parts = _sc_loss(pred, stub_label_map)
    total = parts[0, 0] + parts[1, 0]
    n = parts[0, 1] + parts[1, 1]
    return jnp.where(n > 0, total / jnp.maximum(n, 1.0), 0.0).reshape(())
